# Initial kernel scaffold; baseline (speedup 1.0000x reference)
import jax
import jax.numpy as jnp
from jax.experimental import pallas as pl


def kernel(x, adj, t, W1, b1, W2, b2, W3, b3, W4, b4, fw1, fb1, fw2, fb2, fw3, fb3):
    raise NotImplementedError("write your pallas kernel here")


# speedup vs baseline: 6.0441x; 6.0441x over previous
import functools

import jax
import jax.numpy as jnp
from jax import lax
from jax.experimental import pallas as pl
from jax.experimental.pallas import tpu as pltpu
from jax.experimental.pallas import tpu_sc as plsc

N = 10000
E = 320000
D = 128

NC = 2
NS = 16
NW = NC * NS
CHUNK = 128
EPAD = 327680
CPW = EPAD // (NW * CHUNK)
NPAD = 10240
RPS = NPAD // NS

BR = 640
GRID = NPAD // BR

_f32 = jnp.float32


@functools.lru_cache(maxsize=None)
def _make_sc_scatter(F):
  mesh = plsc.VectorSubcoreMesh(core_axis_name="c", subcore_axis_name="s", num_cores=NC, num_subcores=NS)

  @functools.partial(
      pl.kernel,
      out_type=jax.ShapeDtypeStruct((NC, NPAD, F), _f32),
      mesh=mesh,
      scratch_types=[
          pltpu.VMEM((CPW, CHUNK), jnp.int32),
          pltpu.VMEM((CPW, CHUNK), jnp.int32),
          pltpu.VMEM((CHUNK, F), _f32),
          pltpu.VMEM_SHARED((NPAD, F), _f32),
          pltpu.SemaphoreType.DMA,
      ],
      compiler_params=pltpu.CompilerParams(use_tc_tiling_on_sc=False),
  )
  def k(h_hbm, srcp_hbm, dstp_hbm, zer_hbm, out_hbm, src_v, dst_v, rows_v,
        acc, sem):
    c = lax.axis_index("c")
    s = lax.axis_index("s")
    wid = s * NC + c
    pltpu.sync_copy(srcp_hbm.at[wid], src_v)
    pltpu.sync_copy(dstp_hbm.at[wid], dst_v)
    pltpu.sync_copy(zer_hbm, acc.at[pl.ds(s * RPS, RPS)])
    plsc.subcore_barrier()

    def body(j, carry):
      pltpu.async_copy(h_hbm.at[src_v.at[j]], rows_v, sem).wait()
      pltpu.sync_copy(rows_v, acc.at[dst_v.at[j]], add=True)
      return carry

    lax.fori_loop(0, CPW, body, 0)
    plsc.subcore_barrier()
    pltpu.sync_copy(acc.at[pl.ds(s * RPS, RPS)],
                    out_hbm.at[c, pl.ds(s * RPS, RPS)])

  return k


_DEGW = 16


@functools.lru_cache(maxsize=None)
def _make_sc_deg():
  mesh = plsc.VectorSubcoreMesh(core_axis_name="c", subcore_axis_name="s", num_cores=NC, num_subcores=NS)

  @functools.partial(
      pl.kernel,
      out_type=jax.ShapeDtypeStruct((NC, NPAD, _DEGW), _f32),
      mesh=mesh,
      scratch_types=[
          pltpu.VMEM((CPW, CHUNK), jnp.int32),
          pltpu.VMEM((CHUNK, _DEGW), _f32),
          pltpu.VMEM_SHARED((NPAD, _DEGW), _f32),
      ],
      compiler_params=pltpu.CompilerParams(use_tc_tiling_on_sc=False),
  )
  def k(dstp_hbm, ones_hbm, zer_hbm, out_hbm, dst_v, ones_v, acc):
    c = lax.axis_index("c")
    s = lax.axis_index("s")
    wid = s * NC + c
    pltpu.sync_copy(dstp_hbm.at[wid], dst_v)
    pltpu.sync_copy(ones_hbm, ones_v)
    pltpu.sync_copy(zer_hbm, acc.at[pl.ds(s * RPS, RPS)])
    plsc.subcore_barrier()

    def body(j, carry):
      pltpu.sync_copy(ones_v, acc.at[dst_v.at[j]], add=True)
      return carry

    lax.fori_loop(0, CPW, body, 0)
    plsc.subcore_barrier()
    pltpu.sync_copy(acc.at[pl.ds(s * RPS, RPS)],
                    out_hbm.at[c, pl.ds(s * RPS, RPS)])

  return k


def _full(shape):
  return pl.BlockSpec(shape, lambda i: tuple(0 for _ in shape))


def _rows(shape):
  if len(shape) == 3:
    return pl.BlockSpec(shape, lambda i: (0, i, 0))
  return pl.BlockSpec(shape, lambda i: (i, 0))


def _tc_pre(x, t, deg_p, w1a, w1b):
  def body(x_ref, t_ref, dp_ref, wa_ref, wb_ref, dis_ref, hs_ref):
    dp = dp_ref[...]
    deg = dp[0][:, 0:1] + dp[1][:, 0:1] + 1.0
    dis = lax.rsqrt(deg)
    h = jnp.dot(x_ref[...], wa_ref[...], preferred_element_type=_f32)
    h = h + t_ref[...] * wb_ref[...]
    dis_ref[...] = dis
    hs_ref[...] = dis * h

  return pl.pallas_call(
      body,
      grid=(GRID,),
      in_specs=[
          _rows((BR, D)),
          _rows((BR, 1)),
          _rows((NC, BR, _DEGW)),
          _full((D, 64)),
          _full((1, 64)),
      ],
      out_specs=[_rows((BR, 1)), _rows((BR, 64))],
      out_shape=[
          jax.ShapeDtypeStruct((NPAD, 1), _f32),
          jax.ShapeDtypeStruct((NPAD, 64), _f32),
      ],
  )(x, t, deg_p, w1a, w1b)


def _tc_dense(parts, dis, b, w, fouts):
  fins = [hs.shape[1] for _, hs in parts]
  fin = sum(fins)
  fout = sum(fouts)
  n_parts = len(parts)

  def body(*refs):
    in_refs = refs[:2 * n_parts]
    dis_ref, b_ref, w_ref = refs[2 * n_parts:2 * n_parts + 3]
    out_refs = refs[2 * n_parts + 3:]
    dis = dis_ref[...]
    segs = []
    for i in range(n_parts):
      p = in_refs[2 * i][...]
      hs = in_refs[2 * i + 1][...]
      segs.append(p[0] + p[1] + hs)
    agg = segs[0] if n_parts == 1 else jnp.concatenate(segs, axis=1)
    a = jax.nn.gelu(dis * agg + b_ref[...])
    h = jnp.dot(a, w_ref[...], preferred_element_type=_f32)
    hs_out = dis * h
    off = 0
    for r, f in zip(out_refs, fouts):
      r[...] = hs_out[:, off:off + f]
      off += f

  in_specs = []
  args = []
  for p, hs in parts:
    f = hs.shape[1]
    in_specs += [_rows((NC, BR, f)), _rows((BR, f))]
    args += [p, hs]
  in_specs += [_rows((BR, 1)), _full((1, fin)), _full((fin, fout))]
  args += [dis, b, w]

  return pl.pallas_call(
      body,
      grid=(GRID,),
      in_specs=in_specs,
      out_specs=[_rows((BR, f)) for f in fouts],
      out_shape=[jax.ShapeDtypeStruct((NPAD, f), _f32) for f in fouts],
  )(*args)


def _tc_final(p4, hs4, dis, b4, fw1, fb1, fw2, fb2, fw3, fb3):
  def body(p_ref, hs_ref, dis_ref, b_ref, w1_ref, c1_ref, w2_ref, c2_ref,
           w3_ref, c3_ref, out_ref):
    dis = dis_ref[...]
    p = p_ref[...]
    a = jax.nn.gelu(dis * (p[0] + p[1] + hs_ref[...]) + b_ref[...])
    z = jax.nn.gelu(
        jnp.dot(a, w1_ref[...], preferred_element_type=_f32) + c1_ref[...])
    z = jax.nn.gelu(
        jnp.dot(z, w2_ref[...], preferred_element_type=_f32) + c2_ref[...])
    out_ref[...] = (
        jnp.dot(z, w3_ref[...], preferred_element_type=_f32) + c3_ref[...])

  return pl.pallas_call(
      body,
      grid=(GRID,),
      in_specs=[
          _rows((NC, BR, 128)),
          _rows((BR, 128)),
          _rows((BR, 1)),
          _full((1, 128)),
          _full((128, 256)),
          _full((1, 256)),
          _full((256, 128)),
          _full((1, 128)),
          _full((128, 128)),
          _full((1, 128)),
      ],
      out_specs=_rows((BR, 128)),
      out_shape=jax.ShapeDtypeStruct((NPAD, 128), _f32),
  )(p4, hs4, dis, b4, fw1, fb1, fw2, fb2, fw3, fb3)


def kernel(x, adj, t, W1, b1, W2, b2, W3, b3, W4, b4,
           fw1, fb1, fw2, fb2, fw3, fb3):
  pad_i = jnp.full((EPAD - E,), N, dtype=jnp.int32)
  srcp = jnp.concatenate([adj[0], pad_i]).reshape(NW, CPW, CHUNK)
  dstp = jnp.concatenate([adj[1], pad_i]).reshape(NW, CPW, CHUNK)

  ones_w = jnp.ones((CHUNK, _DEGW), _f32)
  zer_w = jnp.zeros((RPS, _DEGW), _f32)
  zer64 = jnp.zeros((RPS, 64), _f32)
  zer128 = jnp.zeros((RPS, 128), _f32)

  xp = jnp.zeros((NPAD, D), _f32).at[:N].set(x.astype(_f32))
  tp = jnp.zeros((NPAD, 1), _f32).at[:N, 0].set(t.astype(_f32))

  deg_p = _make_sc_deg()(dstp, ones_w, zer_w)
  dis, hs1 = _tc_pre(xp, tp, deg_p, W1[:D], W1[D:].reshape(1, 64))

  p1 = _make_sc_scatter(64)(hs1, srcp, dstp, zer64)
  hs2, = _tc_dense([(p1, hs1)], dis, b1.reshape(1, 64), W2, [128])

  p2 = _make_sc_scatter(128)(hs2, srcp, dstp, zer128)
  hs3a, hs3b = _tc_dense([(p2, hs2)], dis, b2.reshape(1, 128), W3,
                         [128, 128])

  p3a = _make_sc_scatter(128)(hs3a, srcp, dstp, zer128)
  p3b = _make_sc_scatter(128)(hs3b, srcp, dstp, zer128)
  hs4, = _tc_dense([(p3a, hs3a), (p3b, hs3b)], dis, b3.reshape(1, 256), W4,
                   [128])

  p4 = _make_sc_scatter(128)(hs4, srcp, dstp, zer128)
  out = _tc_final(p4, hs4, dis, b4.reshape(1, 128),
                  fw1, fb1.reshape(1, 256), fw2, fb2.reshape(1, 128),
                  fw3, fb3.reshape(1, 128))
  return out[:N]
